# baseline (speedup 1.0000x reference)
import functools

import jax
import jax.numpy as jnp
import numpy as np
from jax import lax
from jax.experimental import pallas as pl
from jax.experimental.pallas import tpu as pltpu
from jax.experimental.pallas import tpu_sc as plsc

N = 500
HM = 512
S = 64
P = N * (N - 1) // 2
NTILE = 32
TPP = 3904
PHAT = NTILE * TPP
CH = TPP // 16
LMW = 250112
TRASH = 250000
ZSTRIPE = LMW // 16

_mesh = plsc.VectorSubcoreMesh(core_axis_name="c", subcore_axis_name="s")


def _body(j0_hbm, j1_hbm, hm_hbm, ii_hbm, jj_hbm,
          mean_hbm, lm_hbm,
          table_sh, j0v, j1v, iiv, jjv,
          idxb, idxb1, gatb, gatb1, wyb, wyb1, wxb, wxb1,
          meanb, detb, s1b, s2b, zb, sem, sem1):
    cid = lax.axis_index("c")
    sid = lax.axis_index("s")
    base = (cid * 16 + sid) * TPP

    @pl.when(sid == 0)
    def _():
        pltpu.sync_copy(hm_hbm, table_sh)

    pltpu.sync_copy(j0_hbm, j0v)
    pltpu.sync_copy(j1_hbm, j1v)
    pltpu.sync_copy(ii_hbm.at[pl.ds(base, TPP)], iiv)
    pltpu.sync_copy(jj_hbm.at[pl.ds(base, TPP)], jjv)

    z16 = jnp.zeros((16,), jnp.float32)

    def _zfill(k, carry):
        zb[pl.ds(k * 16, 16)] = z16
        return carry

    lax.fori_loop(0, 128, _zfill, 0)
    zbase = cid * LMW + sid * ZSTRIPE
    zoff = 0
    for zn in (2048, 2048, 2048, 2048, 2048, 2048, 2048, 1296):
        pltpu.sync_copy(zb.at[pl.ds(0, zn)], lm_hbm.at[pl.ds(zbase + zoff, zn)])
        zoff += zn

    plsc.subcore_barrier()

    inv63 = jnp.float32(1.0 / 63.0)
    plane = cid * LMW

    def _pairdata(c):
        off = c * 16
        iv = iiv[pl.ds(off, 16)]
        jv = jjv[pl.ds(off, 16)]
        s0 = plsc.load_gather(j0v, [iv])
        s1 = plsc.load_gather(j1v, [iv])
        e0 = plsc.load_gather(j0v, [jv])
        e1 = plsc.load_gather(j1v, [jv])
        return (iv, jv, s0, s1, e0 - s0, e1 - s1)

    def _coords_pass(pd, ib, wb, xb):
        del wb, xb
        _, _, s0, s1, d0, d1 = pd
        sy = d0 * inv63
        sx = d1 * inv63

        @plsc.parallel_loop(0, S, unroll=8, carry=(s0, s1))
        def _coords(s, carry2):
            y, x = carry2
            y0i = y.astype(jnp.int32)
            x0i = x.astype(jnp.int32)
            y1i = jnp.minimum(y0i + 1, HM - 1)
            o = s * 16
            ib[pl.ds(o, 16)] = (y0i << 9) + x0i
            ib[pl.ds(1024 + o, 16)] = (y1i << 9) + x0i
            return (y + sy, x + sx)

    def _finish_pass(c, pd, gb, wb, xb):
        off = c * 16
        iv, jv, s0, s1, d0, d1 = pd

        del wb, xb
        sy = d0 * inv63
        sx = d1 * inv63

        @plsc.parallel_loop(
            0, S, unroll=8,
            carry=(jnp.zeros((16,), jnp.float32),
                   jnp.full((16,), 1e9, jnp.float32), s0, s1))
        def _bilin(s, carry2):
            acc, mn, y, x = carry2
            o = s * 16
            w0 = plsc.bitcast(gb[pl.ds(o, 16)], jnp.int32)
            w1 = plsc.bitcast(gb[pl.ds(1024 + o, 16)], jnp.int32)
            v00 = plsc.bitcast(w0 << 16, jnp.float32)
            v01 = plsc.bitcast(w0 & jnp.int32(-65536), jnp.float32)
            v10 = plsc.bitcast(w1 << 16, jnp.float32)
            v11 = plsc.bitcast(w1 & jnp.int32(-65536), jnp.float32)
            wy = y - y.astype(jnp.int32).astype(jnp.float32)
            wx = x - x.astype(jnp.int32).astype(jnp.float32)
            f0 = v00 + wx * (v01 - v00)
            f1 = v10 + wx * (v11 - v10)
            feat = f0 + wy * (f1 - f0)
            return (acc + feat, jnp.minimum(mn, feat), y + sy, x + sx)

        acc, mn, _, _ = _bilin
        mean = acc * jnp.float32(1.0 / 64.0)
        passv = (mean >= 0.5) & (mn > 0.5)

        l2c = jnp.maximum(d0 * d0 + d1 * d1, 1e-8)
        sdotd = s0 * d0 + s1 * d1
        c0 = s0 * d1 - s1 * d0
        lo = 1e-3 * l2c
        hi = (1.0 - 1e-3) * l2c
        t9 = 9.0 * l2c

        def _supp(_):
            def _nloop(n, sp):
                nn = jnp.full((16,), n, jnp.int32)
                p0 = plsc.load_gather(j0v, [nn])
                p1 = plsc.load_gather(j1v, [nn])
                num = p0 * d0 + p1 * d1 - sdotd
                cr = p0 * d1 - p1 * d0 - c0
                ok = ((cr * cr < t9) & (num > lo) & (num < hi)
                      & (nn != iv) & (nn != jv))
                return jnp.where(ok, 1.0, sp)

            return lax.fori_loop(0, N, _nloop, jnp.zeros((16,), jnp.float32))

        suppv = lax.cond(jnp.any(passv), _supp,
                         lambda _: jnp.zeros((16,), jnp.float32), 0)
        det = jnp.where(passv & (suppv == 0.0), 1.0, 0.0)

        meanb[pl.ds(off, 16)] = mean
        detb[pl.ds(off, 16)] = det
        pg = base + off + lax.iota(jnp.int32, 16)
        valid = pg < P
        s1b[pl.ds(off, 16)] = jnp.where(valid, plane + iv * N + jv, plane + TRASH)
        s2b[pl.ds(off, 16)] = jnp.where(valid, plane + jv * N + iv, plane + TRASH)

    def _fire(ib, gb, sm):
        return pltpu.async_copy(table_sh.at[ib], gb, sm)

    def _pipe(g, pd0):
        c0 = 2 * g
        c1 = 2 * g + 1
        c2 = jnp.minimum(2 * g + 2, CH - 1)
        pd1 = _pairdata(c1)
        _coords_pass(pd1, idxb1, wyb1, wxb1)
        cp1 = _fire(idxb1, gatb1, sem1)
        pltpu.make_async_copy(table_sh.at[idxb], gatb, sem).wait()
        _finish_pass(c0, pd0, gatb, wyb, wxb)
        pd2 = _pairdata(c2)
        _coords_pass(pd2, idxb, wyb, wxb)
        _fire(idxb, gatb, sem)
        cp1.wait()
        _finish_pass(c1, pd1, gatb1, wyb1, wxb1)
        return pd2

    pd0 = _pairdata(0)
    _coords_pass(pd0, idxb, wyb, wxb)
    _fire(idxb, gatb, sem)
    pdl = lax.fori_loop(0, CH // 2, _pipe, pd0)
    pltpu.make_async_copy(table_sh.at[idxb], gatb, sem).wait()

    pltpu.sync_copy(meanb, mean_hbm.at[pl.ds(base, TPP)])
    pltpu.async_copy(detb, lm_hbm.at[s1b], sem).wait()
    pltpu.async_copy(detb, lm_hbm.at[s2b], sem).wait()


_sc_call = pl.kernel(
    _body,
    out_type=[
        jax.ShapeDtypeStruct((PHAT,), jnp.float32),
        jax.ShapeDtypeStruct((2 * LMW,), jnp.float32),
    ],
    mesh=_mesh,
    compiler_params=pltpu.CompilerParams(needs_layout_passes=False),
    scratch_types=[
        pltpu.VMEM_SHARED((HM * HM,), jnp.float32),
        pltpu.VMEM((512,), jnp.float32),
        pltpu.VMEM((512,), jnp.float32),
        pltpu.VMEM((TPP,), jnp.int32),
        pltpu.VMEM((TPP,), jnp.int32),
        pltpu.VMEM((2048,), jnp.int32),
        pltpu.VMEM((2048,), jnp.int32),
        pltpu.VMEM((2048,), jnp.float32),
        pltpu.VMEM((2048,), jnp.float32),
        pltpu.VMEM((1024,), jnp.float32),
        pltpu.VMEM((1024,), jnp.float32),
        pltpu.VMEM((1024,), jnp.float32),
        pltpu.VMEM((1024,), jnp.float32),
        pltpu.VMEM((TPP,), jnp.float32),
        pltpu.VMEM((TPP,), jnp.float32),
        pltpu.VMEM((TPP,), jnp.int32),
        pltpu.VMEM((TPP,), jnp.int32),
        pltpu.VMEM((2048,), jnp.float32),
        pltpu.SemaphoreType.DMA,
        pltpu.SemaphoreType.DMA,
    ],
)

_iu, _ju = np.triu_indices(N, k=1)
_II = np.zeros((PHAT,), np.int32)
_JJ = np.ones((PHAT,), np.int32)
_II[:P] = _iu
_JJ[:P] = _ju


def kernel(junctions, heatmap):
    j0 = jnp.zeros((512,), jnp.float32).at[:N].set(junctions[:, 0])
    j1 = jnp.zeros((512,), jnp.float32).at[:N].set(junctions[:, 1])
    hmb = heatmap.reshape(HM * HM).astype(jnp.bfloat16)
    nxt = jnp.concatenate([hmb[1:], hmb[-1:]])
    hmf = jax.lax.bitcast_convert_type(
        jnp.stack([hmb, nxt], axis=-1), jnp.float32)
    mean_pad, lm_pad = _sc_call(j0, j1, hmf,
                                jnp.asarray(_II), jnp.asarray(_JJ))
    seg_mean = mean_pad[:P]
    lm = (lm_pad[:N * N] + lm_pad[LMW:LMW + N * N]).reshape(N, N)
    return lm, seg_mean


# speedup vs baseline: 1.0045x; 1.0045x over previous
import functools

import jax
import jax.numpy as jnp
import numpy as np
from jax import lax
from jax.experimental import pallas as pl
from jax.experimental.pallas import tpu as pltpu
from jax.experimental.pallas import tpu_sc as plsc

N = 500
HM = 512
S = 64
P = N * (N - 1) // 2
NTILE = 32
TPP = 3904
PHAT = NTILE * TPP
CH = TPP // 16
LMW = 250112
TRASH = 250000
ZSTRIPE = LMW // 16

_mesh = plsc.VectorSubcoreMesh(core_axis_name="c", subcore_axis_name="s")


def _body(j0_hbm, j1_hbm, hm_hbm, ii_hbm, jj_hbm,
          mean_hbm, lm_hbm,
          table_sh, j0v, j1v, iiv, jjv,
          idxb, idxb1, gatb, gatb1,
          meanb, detb, s1b, s2b, zb, sem, sem1):
    cid = lax.axis_index("c")
    sid = lax.axis_index("s")
    base = (cid * 16 + sid) * TPP

    @pl.when(sid == 0)
    def _():
        pltpu.sync_copy(hm_hbm, table_sh)

    pltpu.sync_copy(j0_hbm, j0v)
    pltpu.sync_copy(j1_hbm, j1v)
    pltpu.sync_copy(ii_hbm.at[pl.ds(base, TPP)], iiv)
    pltpu.sync_copy(jj_hbm.at[pl.ds(base, TPP)], jjv)

    z16 = jnp.zeros((16,), jnp.float32)

    def _zfill(k, carry):
        zb[pl.ds(k * 16, 16)] = z16
        return carry

    lax.fori_loop(0, 128, _zfill, 0)
    zbase = cid * LMW + sid * ZSTRIPE
    zoff = 0
    for zn in (2048, 2048, 2048, 2048, 2048, 2048, 2048, 1296):
        pltpu.sync_copy(zb.at[pl.ds(0, zn)], lm_hbm.at[pl.ds(zbase + zoff, zn)])
        zoff += zn

    plsc.subcore_barrier()

    inv63 = jnp.float32(1.0 / 63.0)
    plane = cid * LMW

    def _pairdata(c):
        off = c * 16
        iv = iiv[pl.ds(off, 16)]
        jv = jjv[pl.ds(off, 16)]
        s0 = plsc.load_gather(j0v, [iv])
        s1 = plsc.load_gather(j1v, [iv])
        e0 = plsc.load_gather(j0v, [jv])
        e1 = plsc.load_gather(j1v, [jv])
        return (iv, jv, s0, s1, e0 - s0, e1 - s1)

    def _coords_pass(pd, ib, half):
        _, _, s0, s1, d0, d1 = pd
        hb = half * 2048
        sy = d0 * inv63
        sx = d1 * inv63

        def _coords(s, carry2):
            y, x = carry2
            y0i = y.astype(jnp.int32)
            x0i = x.astype(jnp.int32)
            y1i = jnp.minimum(y0i + 1, HM - 1)
            o = hb + s * 16
            ib[pl.ds(o, 16)] = (y0i << 9) + x0i
            ib[pl.ds(1024 + o, 16)] = (y1i << 9) + x0i
            return (y + sy, x + sx)

        lax.fori_loop(0, S, _coords, (s0, s1), unroll=8)

    def _finish_pass(c, pd, gb, half):
        off = c * 16
        iv, jv, s0, s1, d0, d1 = pd
        hb = half * 2048
        sy = d0 * inv63
        sx = d1 * inv63

        def _bilin(s, carry2):
            acc, mn, y, x = carry2
            o = hb + s * 16
            w0 = plsc.bitcast(gb[pl.ds(o, 16)], jnp.int32)
            w1 = plsc.bitcast(gb[pl.ds(1024 + o, 16)], jnp.int32)
            v00 = plsc.bitcast(w0 << 16, jnp.float32)
            v01 = plsc.bitcast(w0 & jnp.int32(-65536), jnp.float32)
            v10 = plsc.bitcast(w1 << 16, jnp.float32)
            v11 = plsc.bitcast(w1 & jnp.int32(-65536), jnp.float32)
            wy = y - y.astype(jnp.int32).astype(jnp.float32)
            wx = x - x.astype(jnp.int32).astype(jnp.float32)
            f0 = v00 + wx * (v01 - v00)
            f1 = v10 + wx * (v11 - v10)
            feat = f0 + wy * (f1 - f0)
            return (acc + feat, jnp.minimum(mn, feat), y + sy, x + sx)

        acc, mn, _, _ = lax.fori_loop(
            0, S, _bilin,
            (jnp.zeros((16,), jnp.float32), jnp.full((16,), 1e9, jnp.float32),
             s0, s1),
            unroll=8)
        mean = acc * jnp.float32(1.0 / 64.0)
        passv = (mean >= 0.5) & (mn > 0.5)

        l2c = jnp.maximum(d0 * d0 + d1 * d1, 1e-8)
        sdotd = s0 * d0 + s1 * d1
        c0 = s0 * d1 - s1 * d0
        lo = 1e-3 * l2c
        hi = (1.0 - 1e-3) * l2c
        t9 = 9.0 * l2c

        def _supp(_):
            def _nloop(n, sp):
                nn = jnp.full((16,), n, jnp.int32)
                p0 = plsc.load_gather(j0v, [nn])
                p1 = plsc.load_gather(j1v, [nn])
                num = p0 * d0 + p1 * d1 - sdotd
                cr = p0 * d1 - p1 * d0 - c0
                ok = ((cr * cr < t9) & (num > lo) & (num < hi)
                      & (nn != iv) & (nn != jv))
                return jnp.where(ok, 1.0, sp)

            return lax.fori_loop(0, N, _nloop, jnp.zeros((16,), jnp.float32))

        suppv = lax.cond(jnp.any(passv), _supp,
                         lambda _: jnp.zeros((16,), jnp.float32), 0)
        det = jnp.where(passv & (suppv == 0.0), 1.0, 0.0)

        meanb[pl.ds(off, 16)] = mean
        detb[pl.ds(off, 16)] = det
        pg = base + off + lax.iota(jnp.int32, 16)
        valid = pg < P
        s1b[pl.ds(off, 16)] = jnp.where(valid, plane + iv * N + jv, plane + TRASH)
        s2b[pl.ds(off, 16)] = jnp.where(valid, plane + jv * N + iv, plane + TRASH)

    def _fire(ib, gb, sm):
        return pltpu.async_copy(table_sh.at[ib], gb, sm)

    def _pipe(g, pds):
        pd0, pd1 = pds
        c0 = 4 * g
        pd2 = _pairdata(c0 + 2)
        pd3 = _pairdata(c0 + 3)
        _coords_pass(pd2, idxb1, 0)
        _coords_pass(pd3, idxb1, 1)
        cpb = _fire(idxb1, gatb1, sem1)
        pltpu.make_async_copy(table_sh.at[idxb], gatb, sem).wait()
        _finish_pass(c0, pd0, gatb, 0)
        _finish_pass(c0 + 1, pd1, gatb, 1)
        pd4 = _pairdata(jnp.minimum(c0 + 4, CH - 1))
        pd5 = _pairdata(jnp.minimum(c0 + 5, CH - 1))
        _coords_pass(pd4, idxb, 0)
        _coords_pass(pd5, idxb, 1)
        _fire(idxb, gatb, sem)
        cpb.wait()
        _finish_pass(c0 + 2, pd2, gatb1, 0)
        _finish_pass(c0 + 3, pd3, gatb1, 1)
        return (pd4, pd5)

    pd0 = _pairdata(0)
    pd1 = _pairdata(1)
    _coords_pass(pd0, idxb, 0)
    _coords_pass(pd1, idxb, 1)
    _fire(idxb, gatb, sem)
    pdl = lax.fori_loop(0, CH // 4, _pipe, (pd0, pd1))
    pltpu.make_async_copy(table_sh.at[idxb], gatb, sem).wait()

    pltpu.sync_copy(meanb, mean_hbm.at[pl.ds(base, TPP)])
    pltpu.async_copy(detb, lm_hbm.at[s1b], sem).wait()
    pltpu.async_copy(detb, lm_hbm.at[s2b], sem).wait()


_sc_call = pl.kernel(
    _body,
    out_type=[
        jax.ShapeDtypeStruct((PHAT,), jnp.float32),
        jax.ShapeDtypeStruct((2 * LMW,), jnp.float32),
    ],
    mesh=_mesh,
    compiler_params=pltpu.CompilerParams(needs_layout_passes=False),
    scratch_types=[
        pltpu.VMEM_SHARED((HM * HM,), jnp.float32),
        pltpu.VMEM((512,), jnp.float32),
        pltpu.VMEM((512,), jnp.float32),
        pltpu.VMEM((TPP,), jnp.int32),
        pltpu.VMEM((TPP,), jnp.int32),
        pltpu.VMEM((4096,), jnp.int32),
        pltpu.VMEM((4096,), jnp.int32),
        pltpu.VMEM((4096,), jnp.float32),
        pltpu.VMEM((4096,), jnp.float32),
        pltpu.VMEM((TPP,), jnp.float32),
        pltpu.VMEM((TPP,), jnp.float32),
        pltpu.VMEM((TPP,), jnp.int32),
        pltpu.VMEM((TPP,), jnp.int32),
        pltpu.VMEM((2048,), jnp.float32),
        pltpu.SemaphoreType.DMA,
        pltpu.SemaphoreType.DMA,
    ],
)

_iu, _ju = np.triu_indices(N, k=1)
_II = np.zeros((PHAT,), np.int32)
_JJ = np.ones((PHAT,), np.int32)
_II[:P] = _iu
_JJ[:P] = _ju


def kernel(junctions, heatmap):
    j0 = jnp.zeros((512,), jnp.float32).at[:N].set(junctions[:, 0])
    j1 = jnp.zeros((512,), jnp.float32).at[:N].set(junctions[:, 1])
    hmb = heatmap.reshape(HM * HM).astype(jnp.bfloat16)
    nxt = jnp.concatenate([hmb[1:], hmb[-1:]])
    hmf = jax.lax.bitcast_convert_type(
        jnp.stack([hmb, nxt], axis=-1), jnp.float32)
    mean_pad, lm_pad = _sc_call(j0, j1, hmf,
                                jnp.asarray(_II), jnp.asarray(_JJ))
    seg_mean = mean_pad[:P]
    lm = (lm_pad[:N * N] + lm_pad[LMW:LMW + N * N]).reshape(N, N)
    return lm, seg_mean
